# baseline (speedup 1.0000x reference)
import functools

import jax
import jax.numpy as jnp
from jax import lax
from jax.experimental import pallas as pl
from jax.experimental.pallas import tpu as pltpu
from jax.experimental.pallas import tpu_sc as plsc

B = 16384
RANK = 16

_info = plsc.get_sparse_core_info()
NC = _info.num_cores
NS = _info.num_subcores
L = _info.num_lanes
NW = NC * NS
BPW = B // NW
CHUNK = 128
NCHUNK = BPW // CHUNK
BLKS = CHUNK // L

_mesh = plsc.VectorSubcoreMesh(core_axis_name="c", subcore_axis_name="s")


@functools.partial(
    pl.kernel,
    mesh=_mesh,
    out_type=(
        jax.ShapeDtypeStruct((B,), jnp.float32),
        jax.ShapeDtypeStruct((B,), jnp.float32),
    ),
    scratch_types=[
        pltpu.VMEM((BPW,), jnp.int32),
        pltpu.VMEM((BPW,), jnp.int32),
        pltpu.VMEM((BPW,), jnp.int32),
        pltpu.VMEM((RANK, CHUNK), jnp.float32),
        pltpu.VMEM((RANK, CHUNK), jnp.float32),
        pltpu.VMEM((RANK, CHUNK), jnp.float32),
        pltpu.VMEM((RANK, CHUNK), jnp.float32),
        pltpu.VMEM((RANK, CHUNK), jnp.float32),
        pltpu.VMEM((RANK, CHUNK), jnp.float32),
        pltpu.VMEM((BPW,), jnp.float32),
        pltpu.VMEM((BPW,), jnp.float32),
        pltpu.SemaphoreType.DMA,
        pltpu.SemaphoreType.DMA,
    ],
    compiler_params=pltpu.CompilerParams(
        needs_layout_passes=False, use_tc_tiling_on_sc=False),
)
def _bpr_sc(ut_hbm, it_hbm, uids_hbm, pids_hbm, nids_hbm,
            outp_hbm, outn_hbm,
            uidx_v, pidx_v, nidx_v,
            ua_v, ub_v, pa_v, pb_v, na_v, nb_v,
            outp_v, outn_v, semA, semB):
    wid = lax.axis_index("s") * NC + lax.axis_index("c")
    sl = pl.ds(wid * BPW, BPW)

    pltpu.sync_copy(uids_hbm.at[sl], uidx_v)
    pltpu.sync_copy(pids_hbm.at[sl], pidx_v)
    pltpu.sync_copy(nids_hbm.at[sl], nidx_v)

    bufs = [(ua_v, pa_v, na_v, semA), (ub_v, pb_v, nb_v, semB)]

    def fire(j):
        ub, pb, nb, sem = bufs[j % 2]
        s = pl.ds(j * CHUNK, CHUNK)
        cps = []
        for f in range(RANK):
            cps.append(pltpu.async_copy(
                ut_hbm.at[f].at[uidx_v.at[s]], ub.at[f], sem))
            cps.append(pltpu.async_copy(
                it_hbm.at[f].at[pidx_v.at[s]], pb.at[f], sem))
            cps.append(pltpu.async_copy(
                it_hbm.at[f].at[nidx_v.at[s]], nb.at[f], sem))
        return cps

    inflight = fire(0)
    for j in range(NCHUNK):
        nxt = fire(j + 1) if j + 1 < NCHUNK else None
        for c in inflight:
            c.wait()
        inflight = nxt

        ub, pb, nb, _ = bufs[j % 2]

        def blk_body(b, carry):
            base = b * L
            accp = jnp.zeros((L,), jnp.float32)
            accn = jnp.zeros((L,), jnp.float32)
            for f in range(RANK):
                u = ub[f, pl.ds(base, L)]
                p = pb[f, pl.ds(base, L)]
                n = nb[f, pl.ds(base, L)]
                accp = accp + u * p
                accn = accn + u * n
            gb = j * CHUNK + base
            outp_v[pl.ds(gb, L)] = accp
            outn_v[pl.ds(gb, L)] = accn
            return carry

        lax.fori_loop(0, BLKS, blk_body, 0)

    pltpu.sync_copy(outp_v, outp_hbm.at[sl])
    pltpu.sync_copy(outn_v, outn_hbm.at[sl])


def kernel(user_ids, pos_items, neg_items, user_emb, item_emb):
    return _bpr_sc(user_emb.T, item_emb.T,
                   user_ids.astype(jnp.int32),
                   pos_items.astype(jnp.int32),
                   neg_items.astype(jnp.int32))
